# baseline (speedup 1.0000x reference)
import functools

import jax
import jax.numpy as jnp
from jax import lax
from jax.experimental import pallas as pl
from jax.experimental.pallas import tpu as pltpu
from jax.experimental.pallas import tpu_sc as plsc

N = 10000
E = 320000
D = 128

NCORES = 2
NSUB = 16
NW = NCORES * NSUB
L = 16

NPT = 320
NPAD = NPT * NW
CH = 6400
NCHUNK = E // CH
BUFW = 12288
FLUSH = 4096
GRP = 64
EPAD = E + GRP

SRC_BITS = 14
SRC_MASK = (1 << SRC_BITS) - 1
SENT_WORD = NPT << SRC_BITS


def _m8(v):
    return pl.multiple_of(v, 8)


def _wid():
    return lax.axis_index("s") * NCORES + lax.axis_index("c")


def _sc_body(build, hp_hbm, *refs):
    if build:
        (src_hbm, dst_hbm, agg_out, packed_ref, counts_ref,
         buf_v, src_v, dst_v, *rest) = refs
    else:
        (packed_ref, counts_ref, agg_out, *rest) = refs
    aggs = rest[:8]
    agg_full, stage_v, words_v, idx_v, cnt_v, sem = rest[8:]

    wid = _wid()

    if build:
        lo = wid * NPT
        hi = lo + NPT

        def chunk(c, carry):
            wpos_v, flushbase = carry
            pltpu.sync_copy(src_hbm.at[pl.ds(_m8(c * CH), CH)], src_v)
            pltpu.sync_copy(dst_hbm.at[pl.ds(_m8(c * CH), CH)], dst_v)

            def step(i, wpos_v):
                sv = src_v[pl.ds(i * L, L)]
                dv = dst_v[pl.ds(i * L, L)]
                m = (dv >= lo) & (dv < hi)
                w = sv | ((dv - lo) << SRC_BITS)
                csum = plsc.cumsum(jnp.where(m, 1, 0).astype(jnp.int32))
                pos = wpos_v + csum - 1
                plsc.store_scatter(buf_v, [pos], w, mask=m)
                return wpos_v + plsc.all_reduce_population_count(m)

            wpos_v = lax.fori_loop(0, CH // L, step, wpos_v)
            wpos = jnp.max(wpos_v)
            nflush = wpos // FLUSH

            def flushk(k, _):
                pltpu.sync_copy(
                    buf_v.at[pl.ds(_m8(k * FLUSH), FLUSH)],
                    packed_ref.at[pl.ds(_m8(wid * EPAD + flushbase + k * FLUSH),
                                        FLUSH)])
                return 0
            lax.fori_loop(0, nflush, flushk, 0)

            @pl.when(nflush > 0)
            def _shift():
                def mv(i, _):
                    buf_v[pl.ds(i * L, L)] = (
                        buf_v[pl.ds(nflush * FLUSH + i * L, L)])
                    return 0
                lax.fori_loop(0, FLUSH // L, mv, 0)

            wpos = wpos - nflush * FLUSH
            return (jnp.full((L,), wpos, jnp.int32),
                    flushbase + nflush * FLUSH)

        wpos_v, flushbase = lax.fori_loop(
            0, NCHUNK, chunk, (jnp.zeros((L,), jnp.int32), jnp.int32(0)))
        wpos = jnp.max(wpos_v)
        k_cnt = flushbase + wpos

        iota = lax.iota(jnp.int32, L)
        sent = jnp.full((L,), SENT_WORD, jnp.int32)
        for k in range(GRP // L):
            plsc.store_scatter(buf_v, [wpos + k * L + iota], sent)
        n64 = (wpos + GRP - 1) // GRP

        def tailk(k, _):
            pltpu.sync_copy(
                buf_v.at[pl.ds(_m8(k * GRP), GRP)],
                packed_ref.at[pl.ds(_m8(wid * EPAD + flushbase + k * GRP), GRP)])
            return 0
        lax.fori_loop(0, n64, tailk, 0)

        cnt_v[...] = jnp.full((L,), k_cnt, jnp.int32)
        pltpu.sync_copy(cnt_v, counts_ref.at[pl.ds(_m8(wid * L), L)])

    zv = jnp.zeros((L,), jnp.float32)

    def zrow(i, _):
        for c in range(D // L):
            aggs[c][pl.ds(i * L, L)] = zv
        return 0
    lax.fori_loop(0, NPT + 1, zrow, 0)

    pltpu.sync_copy(counts_ref.at[pl.ds(_m8(wid * L), L)], cnt_v)
    k_cnt = jnp.max(cnt_v[...])
    n_grp = (k_cnt + GRP - 1) // GRP

    def group(g, _):
        pltpu.sync_copy(packed_ref.at[pl.ds(_m8(wid * EPAD + g * GRP), GRP)],
                        words_v)
        for k in range(GRP // L):
            w = words_v[pl.ds(k * L, L)]
            idx_v[pl.ds(k * L, L)] = w & SRC_MASK
        pltpu.async_copy(hp_hbm.at[idx_v], stage_v, sem).wait()

        def quarter(k, _):
            wv = words_v[pl.ds(k * L, L)]
            dvec = lax.shift_right_logical(wv, SRC_BITS)
            for j in range(L):
                d = dvec[j]
                e = k * L + j
                r = pl.ds(d * L, L)
                for c in range(D // L):
                    aggs[c][r] = jnp.maximum(aggs[c][r],
                                             stage_v[e, pl.ds(c * L, L)])
            return 0
        lax.fori_loop(0, GRP // L, quarter, 0)
        return 0
    lax.fori_loop(0, n_grp, group, 0)

    def arow(i, _):
        for c in range(D // L):
            agg_full[i, pl.ds(c * L, L)] = aggs[c][pl.ds(i * L, L)]
        return 0
    lax.fori_loop(0, NPT, arow, 0)

    pltpu.sync_copy(agg_full, agg_out.at[pl.ds(_m8(wid * NPT), NPT)])


_SC_MESH = plsc.VectorSubcoreMesh(core_axis_name="c", subcore_axis_name="s")

_COMMON_SCRATCH = [
    pltpu.VMEM(((NPT + 1) * L,), jnp.float32) for _ in range(D // L)
] + [
    pltpu.VMEM((NPT, D), jnp.float32),
    pltpu.VMEM((GRP, D), jnp.float32),
    pltpu.VMEM((GRP,), jnp.int32),
    pltpu.VMEM((GRP,), jnp.int32),
    pltpu.VMEM((L,), jnp.int32),
    pltpu.SemaphoreType.DMA,
]

_SC_PARAMS = pltpu.CompilerParams(needs_layout_passes=False)

_seg_max_build = functools.partial(
    pl.kernel,
    mesh=_SC_MESH,
    compiler_params=_SC_PARAMS,
    out_type=(
        jax.ShapeDtypeStruct((NPAD, D), jnp.float32),
        jax.ShapeDtypeStruct((NW * EPAD,), jnp.int32),
        jax.ShapeDtypeStruct((NW * L,), jnp.int32),
    ),
    scratch_types=[
        pltpu.VMEM((BUFW,), jnp.int32),
        pltpu.VMEM((CH,), jnp.int32),
        pltpu.VMEM((CH,), jnp.int32),
    ] + _COMMON_SCRATCH,
)(functools.partial(_sc_body, True))

_seg_max_reuse = functools.partial(
    pl.kernel,
    mesh=_SC_MESH,
    compiler_params=_SC_PARAMS,
    out_type=jax.ShapeDtypeStruct((NPAD, D), jnp.float32),
    scratch_types=list(_COMMON_SCRATCH),
)(functools.partial(_sc_body, False))


_ROWS = 1000
_GRID = N // _ROWS


def _mm(a, w):
    return lax.dot_general(a, w, (((1,), (1,)), ((), ())),
                           preferred_element_type=jnp.float32)


def _ln(rst, g, be):
    mu = jnp.mean(rst, axis=-1, keepdims=True)
    var = jnp.mean((rst - mu) ** 2, axis=-1, keepdims=True)
    return (rst - mu) * lax.rsqrt(var + 1e-5) * g + be


def _tc_input_body(x_ref, wi_ref, bi_ref, wp_ref, bp_ref, h_ref, hp_ref):
    h = _mm(x_ref[...], wi_ref[...]) + bi_ref[...]
    h_ref[...] = h
    hp_ref[...] = jax.nn.relu(_mm(h, wp_ref[...]) + bp_ref[...])


def _tc_mid_body(h_ref, agg_ref, ws_ref, bs_ref, wn_ref, bn_ref,
                 g_ref, be_ref, wp_ref, bp_ref, h_out, hp_out):
    rst = (_mm(h_ref[...], ws_ref[...]) + bs_ref[...]
           + _mm(agg_ref[...], wn_ref[...]) + bn_ref[...])
    rst = jax.nn.relu(rst)
    hn = _ln(rst, g_ref[...], be_ref[...])
    h_out[...] = hn
    hp_out[...] = jax.nn.relu(_mm(hn, wp_ref[...]) + bp_ref[...])


def _tc_final_body(h_ref, agg_ref, ws_ref, bs_ref, wn_ref, bn_ref,
                   g_ref, be_ref, o_ref):
    rst = (_mm(h_ref[...], ws_ref[...]) + bs_ref[...]
           + _mm(agg_ref[...], wn_ref[...]) + bn_ref[...])
    o_ref[...] = _ln(rst, g_ref[...], be_ref[...])


_row_spec = pl.BlockSpec((_ROWS, D), lambda i: (i, 0))
_w_spec = pl.BlockSpec((D, D), lambda i: (0, 0))
_b_spec = pl.BlockSpec((1, D), lambda i: (0, 0))
_f32 = jnp.float32

_tc_input = pl.pallas_call(
    _tc_input_body,
    grid=(_GRID,),
    in_specs=[_row_spec, _w_spec, _b_spec, _w_spec, _b_spec],
    out_specs=[_row_spec, _row_spec],
    out_shape=[jax.ShapeDtypeStruct((N, D), _f32)] * 2,
)

_tc_mid = pl.pallas_call(
    _tc_mid_body,
    grid=(_GRID,),
    in_specs=[_row_spec, _row_spec, _w_spec, _b_spec, _w_spec, _b_spec,
              _b_spec, _b_spec, _w_spec, _b_spec],
    out_specs=[_row_spec, _row_spec],
    out_shape=[jax.ShapeDtypeStruct((N, D), _f32)] * 2,
)

_tc_final = pl.pallas_call(
    _tc_final_body,
    grid=(_GRID,),
    in_specs=[_row_spec, _row_spec, _w_spec, _b_spec, _w_spec, _b_spec,
              _b_spec, _b_spec],
    out_specs=_row_spec,
    out_shape=jax.ShapeDtypeStruct((N, D), _f32),
)


def kernel(x, edge_index, W_in, b_in,
           Wp0, bp0, Ws0, bs0, Wn0, bn0, g0, be0,
           Wp1, bp1, Ws1, bs1, Wn1, bn1, g1, be1,
           Wp2, bp2, Ws2, bs2, Wn2, bn2, g2, be2):
    src = edge_index[0]
    dst = edge_index[1]
    r = lambda v: v.reshape(1, D)

    h, hp = _tc_input(x, W_in, r(b_in), Wp0, r(bp0))
    agg, packed, counts = _seg_max_build(hp, src, dst)
    h, hp = _tc_mid(h, agg[:N], Ws0, r(bs0), Wn0, r(bn0),
                    r(g0), r(be0), Wp1, r(bp1))
    agg = _seg_max_reuse(hp, packed, counts)
    h, hp = _tc_mid(h, agg[:N], Ws1, r(bs1), Wn1, r(bn1),
                    r(g1), r(be1), Wp2, r(bp2))
    agg = _seg_max_reuse(hp, packed, counts)
    return _tc_final(h, agg[:N], Ws2, r(bs2), Wn2, r(bn2), r(g2), r(be2))


# speedup vs baseline: 1.5523x; 1.5523x over previous
import functools

import jax
import jax.numpy as jnp
from jax import lax
from jax.experimental import pallas as pl
from jax.experimental.pallas import tpu as pltpu
from jax.experimental.pallas import tpu_sc as plsc

N = 10000
E = 320000
D = 128

NCORES = 2
NSUB = 16
NW = NCORES * NSUB
L = 16

NPT = 320
NPAD = NPT * NW
CH = 6400
NCHUNK = E // CH
BUFW = 12288
FLUSH = 4096
GRP = 64
EPAD = E + GRP

CB = 32
NCB = D // CB

SRC_BITS = 14
SRC_MASK = (1 << SRC_BITS) - 1
SENT_WORD = NPT << SRC_BITS


def _m8(v):
    return pl.multiple_of(v, 8)


def _m16(v):
    return pl.multiple_of(v, 16)


def _wid():
    return lax.axis_index("s") * NCORES + lax.axis_index("c")


def _sc_body(build, hp_hbm, *refs):
    if build:
        (src_hbm, dst_hbm, agg_out, packed_ref, counts_ref,
         buf_v, src_v, dst_v, *rest) = refs
    else:
        (packed_ref, counts_ref, agg_out, *rest) = refs
    aggs = rest[:NCB]
    hp_sh, agg_full, stage_v, words_v, idx_v, cnt_v, sem = rest[NCB:]

    wid = _wid()

    if build:
        lo = wid * NPT
        hi = lo + NPT

        def chunk(c, carry):
            wpos_v, flushbase = carry
            pltpu.sync_copy(src_hbm.at[pl.ds(_m8(c * CH), CH)], src_v)
            pltpu.sync_copy(dst_hbm.at[pl.ds(_m8(c * CH), CH)], dst_v)

            def step(i, wpos_v):
                sv = src_v[pl.ds(i * L, L)]
                dv = dst_v[pl.ds(i * L, L)]
                m = (dv >= lo) & (dv < hi)
                w = sv | ((dv - lo) << SRC_BITS)
                csum = plsc.cumsum(jnp.where(m, 1, 0).astype(jnp.int32))
                pos = wpos_v + csum - 1
                plsc.store_scatter(buf_v, [pos], w, mask=m)
                return wpos_v + plsc.all_reduce_population_count(m)

            wpos_v = lax.fori_loop(0, CH // L, step, wpos_v)
            wpos = jnp.max(wpos_v)
            nflush = wpos // FLUSH

            def flushk(k, _):
                pltpu.sync_copy(
                    buf_v.at[pl.ds(_m8(k * FLUSH), FLUSH)],
                    packed_ref.at[pl.ds(_m8(wid * EPAD + flushbase + k * FLUSH),
                                        FLUSH)])
                return 0
            lax.fori_loop(0, nflush, flushk, 0)

            @pl.when(nflush > 0)
            def _shift():
                def mv(i, _):
                    buf_v[pl.ds(i * L, L)] = (
                        buf_v[pl.ds(nflush * FLUSH + i * L, L)])
                    return 0
                lax.fori_loop(0, FLUSH // L, mv, 0)

            wpos = wpos - nflush * FLUSH
            return (jnp.full((L,), wpos, jnp.int32),
                    flushbase + nflush * FLUSH)

        wpos_v, flushbase = lax.fori_loop(
            0, NCHUNK, chunk, (jnp.zeros((L,), jnp.int32), jnp.int32(0)))
        wpos = jnp.max(wpos_v)
        k_cnt = flushbase + wpos

        iota = lax.iota(jnp.int32, L)
        sent = jnp.full((L,), SENT_WORD, jnp.int32)
        for k in range(GRP // L):
            plsc.store_scatter(buf_v, [wpos + k * L + iota], sent)
        n64 = (wpos + GRP - 1) // GRP

        def tailk(k, _):
            pltpu.sync_copy(
                buf_v.at[pl.ds(_m8(k * GRP), GRP)],
                packed_ref.at[pl.ds(_m8(wid * EPAD + flushbase + k * GRP), GRP)])
            return 0
        lax.fori_loop(0, n64, tailk, 0)

        cnt_v[...] = jnp.full((L,), k_cnt, jnp.int32)
        pltpu.sync_copy(cnt_v, counts_ref.at[pl.ds(_m8(wid * L), L)])

    sid = lax.axis_index("s")

    @pl.when(sid < NSUB - 1)
    def _stage_main():
        off = _m16(sid * 640)
        pltpu.sync_copy(hp_hbm.at[pl.ds(off, 640)],
                        hp_sh.at[pl.ds(off, 640)])

    @pl.when(sid == NSUB - 1)
    def _stage_tail():
        off = _m16(sid * 640)
        pltpu.sync_copy(hp_hbm.at[pl.ds(off, N - 640 * (NSUB - 1))],
                        hp_sh.at[pl.ds(off, N - 640 * (NSUB - 1))])

    zv = jnp.zeros((CB,), jnp.bfloat16)

    def zrow(i, _):
        for c in range(NCB):
            aggs[c][pl.ds(i * CB, CB)] = zv
        return 0
    lax.fori_loop(0, NPT + 1, zrow, 0)
    plsc.subcore_barrier()

    pltpu.sync_copy(counts_ref.at[pl.ds(_m8(wid * L), L)], cnt_v)
    k_cnt = jnp.max(cnt_v[...])
    n_grp = (k_cnt + GRP - 1) // GRP

    def group(g, _):
        pltpu.sync_copy(packed_ref.at[pl.ds(_m8(wid * EPAD + g * GRP), GRP)],
                        words_v)
        for k in range(GRP // L):
            w = words_v[pl.ds(k * L, L)]
            idx_v[pl.ds(k * L, L)] = w & SRC_MASK
        pltpu.async_copy(hp_sh.at[idx_v], stage_v, sem).wait()

        def quarter(k, _):
            wv = words_v[pl.ds(k * L, L)]
            dvec = lax.shift_right_logical(wv, SRC_BITS)
            for j in range(L):
                d = dvec[j]
                e = k * L + j
                r = pl.ds(d * CB, CB)
                for c in range(NCB):
                    sb = plsc.bitcast(stage_v[e, pl.ds(c * L, L)],
                                      jnp.bfloat16)
                    aggs[c][r] = jnp.maximum(aggs[c][r], sb)
            return 0
        lax.fori_loop(0, GRP // L, quarter, 0)
        return 0
    lax.fori_loop(0, n_grp, group, 0)

    def arow(i, _):
        for c in range(NCB):
            agg_full[i, pl.ds(c * CB, CB)] = aggs[c][pl.ds(i * CB, CB)]
        return 0
    lax.fori_loop(0, NPT, arow, 0)

    pltpu.sync_copy(agg_full, agg_out.at[pl.ds(_m16(wid * NPT), NPT)])


_SC_MESH = plsc.VectorSubcoreMesh(core_axis_name="c", subcore_axis_name="s")

_COMMON_SCRATCH = [
    pltpu.VMEM(((NPT + 1) * CB,), jnp.bfloat16) for _ in range(NCB)
] + [
    pltpu.VMEM_SHARED((N, D // 2), jnp.int32),
    pltpu.VMEM((NPT, D), jnp.bfloat16),
    pltpu.VMEM((GRP, D // 2), jnp.int32),
    pltpu.VMEM((GRP,), jnp.int32),
    pltpu.VMEM((GRP,), jnp.int32),
    pltpu.VMEM((L,), jnp.int32),
    pltpu.SemaphoreType.DMA,
]

_SC_PARAMS = pltpu.CompilerParams(needs_layout_passes=False,
                                  use_tc_tiling_on_sc=False)

_seg_max_build = functools.partial(
    pl.kernel,
    mesh=_SC_MESH,
    compiler_params=_SC_PARAMS,
    out_type=(
        jax.ShapeDtypeStruct((NPAD, D), jnp.bfloat16),
        jax.ShapeDtypeStruct((NW * EPAD,), jnp.int32),
        jax.ShapeDtypeStruct((NW * L,), jnp.int32),
    ),
    scratch_types=[
        pltpu.VMEM((BUFW,), jnp.int32),
        pltpu.VMEM((CH,), jnp.int32),
        pltpu.VMEM((CH,), jnp.int32),
    ] + _COMMON_SCRATCH,
)(functools.partial(_sc_body, True))

_seg_max_reuse = functools.partial(
    pl.kernel,
    mesh=_SC_MESH,
    compiler_params=_SC_PARAMS,
    out_type=jax.ShapeDtypeStruct((NPAD, D), jnp.bfloat16),
    scratch_types=list(_COMMON_SCRATCH),
)(functools.partial(_sc_body, False))


_ROWS = 1000
_GRID = N // _ROWS


def _mm(a, w):
    return lax.dot_general(a, w, (((1,), (1,)), ((), ())),
                           preferred_element_type=jnp.float32)


def _ln(rst, g, be):
    mu = jnp.mean(rst, axis=-1, keepdims=True)
    var = jnp.mean((rst - mu) ** 2, axis=-1, keepdims=True)
    return (rst - mu) * lax.rsqrt(var + 1e-5) * g + be


def _tc_input_body(x_ref, wi_ref, bi_ref, wp_ref, bp_ref, h_ref, hp_ref):
    h = _mm(x_ref[...], wi_ref[...]) + bi_ref[...]
    h_ref[...] = h
    hp_ref[...] = jax.nn.relu(
        _mm(h, wp_ref[...]) + bp_ref[...]).astype(jnp.bfloat16)


def _tc_mid_body(h_ref, agg_ref, ws_ref, bs_ref, wn_ref, bn_ref,
                 g_ref, be_ref, wp_ref, bp_ref, h_out, hp_out):
    agg = agg_ref[...].astype(jnp.float32)
    rst = (_mm(h_ref[...], ws_ref[...]) + bs_ref[...]
           + _mm(agg, wn_ref[...]) + bn_ref[...])
    rst = jax.nn.relu(rst)
    hn = _ln(rst, g_ref[...], be_ref[...])
    h_out[...] = hn
    hp_out[...] = jax.nn.relu(
        _mm(hn, wp_ref[...]) + bp_ref[...]).astype(jnp.bfloat16)


def _tc_final_body(h_ref, agg_ref, ws_ref, bs_ref, wn_ref, bn_ref,
                   g_ref, be_ref, o_ref):
    agg = agg_ref[...].astype(jnp.float32)
    rst = (_mm(h_ref[...], ws_ref[...]) + bs_ref[...]
           + _mm(agg, wn_ref[...]) + bn_ref[...])
    o_ref[...] = _ln(rst, g_ref[...], be_ref[...])


_row_spec = pl.BlockSpec((_ROWS, D), lambda i: (i, 0))
_w_spec = pl.BlockSpec((D, D), lambda i: (0, 0))
_b_spec = pl.BlockSpec((1, D), lambda i: (0, 0))
_f32 = jnp.float32

_tc_input = pl.pallas_call(
    _tc_input_body,
    grid=(_GRID,),
    in_specs=[_row_spec, _w_spec, _b_spec, _w_spec, _b_spec],
    out_specs=[_row_spec, _row_spec],
    out_shape=[jax.ShapeDtypeStruct((N, D), _f32),
               jax.ShapeDtypeStruct((N, D), jnp.bfloat16)],
)

_tc_mid = pl.pallas_call(
    _tc_mid_body,
    grid=(_GRID,),
    in_specs=[_row_spec, _row_spec, _w_spec, _b_spec, _w_spec, _b_spec,
              _b_spec, _b_spec, _w_spec, _b_spec],
    out_specs=[_row_spec, _row_spec],
    out_shape=[jax.ShapeDtypeStruct((N, D), _f32),
               jax.ShapeDtypeStruct((N, D), jnp.bfloat16)],
)

_tc_final = pl.pallas_call(
    _tc_final_body,
    grid=(_GRID,),
    in_specs=[_row_spec, _row_spec, _w_spec, _b_spec, _w_spec, _b_spec,
              _b_spec, _b_spec],
    out_specs=_row_spec,
    out_shape=jax.ShapeDtypeStruct((N, D), _f32),
)


def kernel(x, edge_index, W_in, b_in,
           Wp0, bp0, Ws0, bs0, Wn0, bn0, g0, be0,
           Wp1, bp1, Ws1, bs1, Wn1, bn1, g1, be1,
           Wp2, bp2, Ws2, bs2, Wn2, bn2, g2, be2):
    src = edge_index[0]
    dst = edge_index[1]
    r = lambda v: v.reshape(1, D)

    pk = lambda a: lax.bitcast_convert_type(
        a.reshape(N, D // 2, 2), jnp.int32)

    h, hp = _tc_input(x, W_in, r(b_in), Wp0, r(bp0))
    hp = pk(hp)
    agg, packed, counts = _seg_max_build(hp, src, dst)
    h, hp = _tc_mid(h, agg[:N], Ws0, r(bs0), Wn0, r(bn0),
                    r(g0), r(be0), Wp1, r(bp1))
    hp = pk(hp)
    agg = _seg_max_reuse(hp, packed, counts)
    h, hp = _tc_mid(h, agg[:N], Ws1, r(bs1), Wn1, r(bn1),
                    r(g1), r(be1), Wp2, r(bp2))
    hp = pk(hp)
    agg = _seg_max_reuse(hp, packed, counts)
    return _tc_final(h, agg[:N], Ws2, r(bs2), Wn2, r(bn2), r(g2), r(be2))


# speedup vs baseline: 1.6909x; 1.0893x over previous
import functools

import jax
import jax.numpy as jnp
from jax import lax
from jax.experimental import pallas as pl
from jax.experimental.pallas import tpu as pltpu
from jax.experimental.pallas import tpu_sc as plsc

N = 10000
E = 320000
D = 128

NCORES = 2
NSUB = 16
NW = NCORES * NSUB
L = 16

NPT = 320
NPAD = NPT * NW
CH = 6400
NCHUNK = E // CH
BUFW = 12288
FLUSH = 4096
GRP = 64
EPAD = E + GRP

CB = 32
NCB = D // CB

SRC_BITS = 14
SRC_MASK = (1 << SRC_BITS) - 1
SENT_WORD = NPT << SRC_BITS


def _m8(v):
    return pl.multiple_of(v, 8)


def _m16(v):
    return pl.multiple_of(v, 16)


def _wid():
    return lax.axis_index("s") * NCORES + lax.axis_index("c")


def _sc_body(build, hp_hbm, *refs):
    if build:
        (src_hbm, dst_hbm, agg_out, packed_ref, counts_ref,
         buf_v, src_v, dst_v, *rest) = refs
    else:
        (packed_ref, counts_ref, agg_out, *rest) = refs
    aggs = rest[:NCB]
    (hp_sh, agg_full, st0, st1, wd0, wd1, ix0, ix1, cnt_v,
     sm0, sm1) = rest[NCB:]
    stage_v = (st0, st1)
    words_v = (wd0, wd1)
    idx_v = (ix0, ix1)
    sem = (sm0, sm1)

    wid = _wid()

    if build:
        lo = wid * NPT
        hi = lo + NPT

        def chunk(c, carry):
            wpos_v, flushbase = carry
            pltpu.sync_copy(src_hbm.at[pl.ds(_m8(c * CH), CH)], src_v)
            pltpu.sync_copy(dst_hbm.at[pl.ds(_m8(c * CH), CH)], dst_v)

            def step(i, wpos_v):
                sv = src_v[pl.ds(i * L, L)]
                dv = dst_v[pl.ds(i * L, L)]
                m = (dv >= lo) & (dv < hi)
                w = sv | ((dv - lo) << SRC_BITS)
                csum = plsc.cumsum(jnp.where(m, 1, 0).astype(jnp.int32))
                pos = wpos_v + csum - 1
                plsc.store_scatter(buf_v, [pos], w, mask=m)
                return wpos_v + plsc.all_reduce_population_count(m)

            wpos_v = lax.fori_loop(0, CH // L, step, wpos_v)
            wpos = jnp.max(wpos_v)
            nflush = wpos // FLUSH

            def flushk(k, _):
                pltpu.sync_copy(
                    buf_v.at[pl.ds(_m8(k * FLUSH), FLUSH)],
                    packed_ref.at[pl.ds(_m8(wid * EPAD + flushbase + k * FLUSH),
                                        FLUSH)])
                return 0
            lax.fori_loop(0, nflush, flushk, 0)

            @pl.when(nflush > 0)
            def _shift():
                def mv(i, _):
                    buf_v[pl.ds(i * L, L)] = (
                        buf_v[pl.ds(nflush * FLUSH + i * L, L)])
                    return 0
                lax.fori_loop(0, FLUSH // L, mv, 0)

            wpos = wpos - nflush * FLUSH
            return (jnp.full((L,), wpos, jnp.int32),
                    flushbase + nflush * FLUSH)

        wpos_v, flushbase = lax.fori_loop(
            0, NCHUNK, chunk, (jnp.zeros((L,), jnp.int32), jnp.int32(0)))
        wpos = jnp.max(wpos_v)
        k_cnt = flushbase + wpos

        iota = lax.iota(jnp.int32, L)
        sent = jnp.full((L,), SENT_WORD, jnp.int32)
        for k in range(GRP // L):
            plsc.store_scatter(buf_v, [wpos + k * L + iota], sent)
        n64 = (wpos + GRP - 1) // GRP

        def tailk(k, _):
            pltpu.sync_copy(
                buf_v.at[pl.ds(_m8(k * GRP), GRP)],
                packed_ref.at[pl.ds(_m8(wid * EPAD + flushbase + k * GRP), GRP)])
            return 0
        lax.fori_loop(0, n64, tailk, 0)

        cnt_v[...] = jnp.full((L,), k_cnt, jnp.int32)
        pltpu.sync_copy(cnt_v, counts_ref.at[pl.ds(_m8(wid * L), L)])

    sid = lax.axis_index("s")

    @pl.when(sid < NSUB - 1)
    def _stage_main():
        off = _m16(sid * 640)
        pltpu.sync_copy(hp_hbm.at[pl.ds(off, 640)],
                        hp_sh.at[pl.ds(off, 640)])

    @pl.when(sid == NSUB - 1)
    def _stage_tail():
        off = _m16(sid * 640)
        pltpu.sync_copy(hp_hbm.at[pl.ds(off, N - 640 * (NSUB - 1))],
                        hp_sh.at[pl.ds(off, N - 640 * (NSUB - 1))])

    zv = jnp.zeros((CB,), jnp.bfloat16)

    def zrow(i, _):
        for c in range(NCB):
            aggs[c][pl.ds(i * CB, CB)] = zv
        return 0
    lax.fori_loop(0, NPT + 1, zrow, 0)
    plsc.subcore_barrier()

    pltpu.sync_copy(counts_ref.at[pl.ds(_m8(wid * L), L)], cnt_v)
    k_cnt = jnp.max(cnt_v[...])
    n_grp = (k_cnt + GRP - 1) // GRP

    def _launch(b, g):
        pltpu.sync_copy(packed_ref.at[pl.ds(_m8(wid * EPAD + g * GRP), GRP)],
                        words_v[b])
        for k in range(GRP // L):
            w = words_v[b][pl.ds(k * L, L)]
            idx_v[b][pl.ds(k * L, L)] = w & SRC_MASK
        pltpu.make_async_copy(hp_sh.at[idx_v[b]], stage_v[b], sem[b]).start()

    def _process(b):
        pltpu.make_async_copy(hp_sh.at[idx_v[b]], stage_v[b], sem[b]).wait()

        def quarter(k, _):
            wv = words_v[b][pl.ds(k * L, L)]
            dvec = lax.shift_right_logical(wv, SRC_BITS)
            for j in range(L):
                d = dvec[j]
                e = k * L + j
                r = pl.ds(d * CB, CB)
                for c in range(NCB):
                    sb = plsc.bitcast(stage_v[b][e, pl.ds(c * L, L)],
                                      jnp.bfloat16)
                    aggs[c][r] = jnp.maximum(aggs[c][r], sb)
            return 0
        lax.fori_loop(0, GRP // L, quarter, 0)

    @pl.when(n_grp > 0)
    def _pipeline():
        _launch(0, 0)

        def pair(p, _):
            g1 = 2 * p + 1
            g2 = 2 * p + 2

            @pl.when(g1 < n_grp)
            def _():
                _launch(1, g1)
            _process(0)

            @pl.when(g2 < n_grp)
            def _():
                _launch(0, g2)

            @pl.when(g1 < n_grp)
            def _():
                _process(1)
            return 0
        lax.fori_loop(0, (n_grp + 1) // 2, pair, 0)

    def arow(i, _):
        for c in range(NCB):
            agg_full[i, pl.ds(c * CB, CB)] = aggs[c][pl.ds(i * CB, CB)]
        return 0
    lax.fori_loop(0, NPT, arow, 0)

    pltpu.sync_copy(agg_full, agg_out.at[pl.ds(_m16(wid * NPT), NPT)])


_SC_MESH = plsc.VectorSubcoreMesh(core_axis_name="c", subcore_axis_name="s")

_COMMON_SCRATCH = [
    pltpu.VMEM(((NPT + 1) * CB,), jnp.bfloat16) for _ in range(NCB)
] + [
    pltpu.VMEM_SHARED((N, D // 2), jnp.int32),
    pltpu.VMEM((NPT, D), jnp.bfloat16),
    pltpu.VMEM((GRP, D // 2), jnp.int32),
    pltpu.VMEM((GRP, D // 2), jnp.int32),
    pltpu.VMEM((GRP,), jnp.int32),
    pltpu.VMEM((GRP,), jnp.int32),
    pltpu.VMEM((GRP,), jnp.int32),
    pltpu.VMEM((GRP,), jnp.int32),
    pltpu.VMEM((L,), jnp.int32),
    pltpu.SemaphoreType.DMA,
    pltpu.SemaphoreType.DMA,
]

_SC_PARAMS = pltpu.CompilerParams(needs_layout_passes=False,
                                  use_tc_tiling_on_sc=False)

_seg_max_build = functools.partial(
    pl.kernel,
    mesh=_SC_MESH,
    compiler_params=_SC_PARAMS,
    out_type=(
        jax.ShapeDtypeStruct((NPAD, D), jnp.bfloat16),
        jax.ShapeDtypeStruct((NW * EPAD,), jnp.int32),
        jax.ShapeDtypeStruct((NW * L,), jnp.int32),
    ),
    scratch_types=[
        pltpu.VMEM((BUFW,), jnp.int32),
        pltpu.VMEM((CH,), jnp.int32),
        pltpu.VMEM((CH,), jnp.int32),
    ] + _COMMON_SCRATCH,
)(functools.partial(_sc_body, True))

_seg_max_reuse = functools.partial(
    pl.kernel,
    mesh=_SC_MESH,
    compiler_params=_SC_PARAMS,
    out_type=jax.ShapeDtypeStruct((NPAD, D), jnp.bfloat16),
    scratch_types=list(_COMMON_SCRATCH),
)(functools.partial(_sc_body, False))


_ROWS = 1000
_GRID = N // _ROWS


def _mm(a, w):
    return lax.dot_general(a, w, (((1,), (1,)), ((), ())),
                           preferred_element_type=jnp.float32)


def _ln(rst, g, be):
    mu = jnp.mean(rst, axis=-1, keepdims=True)
    var = jnp.mean((rst - mu) ** 2, axis=-1, keepdims=True)
    return (rst - mu) * lax.rsqrt(var + 1e-5) * g + be


def _tc_input_body(x_ref, wi_ref, bi_ref, wp_ref, bp_ref, h_ref, hp_ref):
    h = _mm(x_ref[...], wi_ref[...]) + bi_ref[...]
    h_ref[...] = h
    hp_ref[...] = jax.nn.relu(
        _mm(h, wp_ref[...]) + bp_ref[...]).astype(jnp.bfloat16)


def _tc_mid_body(h_ref, agg_ref, ws_ref, bs_ref, wn_ref, bn_ref,
                 g_ref, be_ref, wp_ref, bp_ref, h_out, hp_out):
    agg = agg_ref[...].astype(jnp.float32)
    rst = (_mm(h_ref[...], ws_ref[...]) + bs_ref[...]
           + _mm(agg, wn_ref[...]) + bn_ref[...])
    rst = jax.nn.relu(rst)
    hn = _ln(rst, g_ref[...], be_ref[...])
    h_out[...] = hn
    hp_out[...] = jax.nn.relu(
        _mm(hn, wp_ref[...]) + bp_ref[...]).astype(jnp.bfloat16)


def _tc_final_body(h_ref, agg_ref, ws_ref, bs_ref, wn_ref, bn_ref,
                   g_ref, be_ref, o_ref):
    agg = agg_ref[...].astype(jnp.float32)
    rst = (_mm(h_ref[...], ws_ref[...]) + bs_ref[...]
           + _mm(agg, wn_ref[...]) + bn_ref[...])
    o_ref[...] = _ln(rst, g_ref[...], be_ref[...])


_row_spec = pl.BlockSpec((_ROWS, D), lambda i: (i, 0))
_w_spec = pl.BlockSpec((D, D), lambda i: (0, 0))
_b_spec = pl.BlockSpec((1, D), lambda i: (0, 0))
_f32 = jnp.float32

_tc_input = pl.pallas_call(
    _tc_input_body,
    grid=(_GRID,),
    in_specs=[_row_spec, _w_spec, _b_spec, _w_spec, _b_spec],
    out_specs=[_row_spec, _row_spec],
    out_shape=[jax.ShapeDtypeStruct((N, D), _f32),
               jax.ShapeDtypeStruct((N, D), jnp.bfloat16)],
)

_tc_mid = pl.pallas_call(
    _tc_mid_body,
    grid=(_GRID,),
    in_specs=[_row_spec, _row_spec, _w_spec, _b_spec, _w_spec, _b_spec,
              _b_spec, _b_spec, _w_spec, _b_spec],
    out_specs=[_row_spec, _row_spec],
    out_shape=[jax.ShapeDtypeStruct((N, D), _f32),
               jax.ShapeDtypeStruct((N, D), jnp.bfloat16)],
)

_tc_final = pl.pallas_call(
    _tc_final_body,
    grid=(_GRID,),
    in_specs=[_row_spec, _row_spec, _w_spec, _b_spec, _w_spec, _b_spec,
              _b_spec, _b_spec],
    out_specs=_row_spec,
    out_shape=jax.ShapeDtypeStruct((N, D), _f32),
)


def kernel(x, edge_index, W_in, b_in,
           Wp0, bp0, Ws0, bs0, Wn0, bn0, g0, be0,
           Wp1, bp1, Ws1, bs1, Wn1, bn1, g1, be1,
           Wp2, bp2, Ws2, bs2, Wn2, bn2, g2, be2):
    src = edge_index[0]
    dst = edge_index[1]
    r = lambda v: v.reshape(1, D)

    pk = lambda a: lax.bitcast_convert_type(
        a.reshape(N, D // 2, 2), jnp.int32)

    h, hp = _tc_input(x, W_in, r(b_in), Wp0, r(bp0))
    hp = pk(hp)
    agg, packed, counts = _seg_max_build(hp, src, dst)
    h, hp = _tc_mid(h, agg[:N], Ws0, r(bs0), Wn0, r(bn0),
                    r(g0), r(be0), Wp1, r(bp1))
    hp = pk(hp)
    agg = _seg_max_reuse(hp, packed, counts)
    h, hp = _tc_mid(h, agg[:N], Ws1, r(bs1), Wn1, r(bn1),
                    r(g1), r(be1), Wp2, r(bp2))
    hp = pk(hp)
    agg = _seg_max_reuse(hp, packed, counts)
    return _tc_final(h, agg[:N], Ws2, r(bs2), Wn2, r(bn2), r(g2), r(be2))


# speedup vs baseline: 1.7325x; 1.0246x over previous
import functools

import jax
import jax.numpy as jnp
from jax import lax
from jax.experimental import pallas as pl
from jax.experimental.pallas import tpu as pltpu
from jax.experimental.pallas import tpu_sc as plsc

N = 10000
E = 320000
D = 128

NCORES = 2
NSUB = 16
NW = NCORES * NSUB
L = 16

NPT = 320
NPAD = NPT * NW
CH = 6400
NCHUNK = E // CH
BUFW = 12288
FLUSH = 4096
GRP = 64
EPAD = E + GRP

CB = 32
NCB = D // CB

SRC_BITS = 14
SRC_MASK = (1 << SRC_BITS) - 1
DST_SHIFT = SRC_BITS
SENT_WORD = (NPT * CB) << DST_SHIFT


def _m8(v):
    return pl.multiple_of(v, 8)


def _m16(v):
    return pl.multiple_of(v, 16)


def _wid():
    return lax.axis_index("s") * NCORES + lax.axis_index("c")


def _build_body(src_hbm, dst_hbm, packed_ref, counts_ref,
                buf_v, src_v, dst_v, cnt_v):
    wid = _wid()
    lo = wid * NPT
    hi = lo + NPT

    def chunk(c, carry):
        wpos_v, flushbase = carry
        pltpu.sync_copy(src_hbm.at[pl.ds(_m8(c * CH), CH)], src_v)
        pltpu.sync_copy(dst_hbm.at[pl.ds(_m8(c * CH), CH)], dst_v)

        def step(i, wpos_v):
            sv = src_v[pl.ds(i * L, L)]
            dv = dst_v[pl.ds(i * L, L)]
            m = (dv >= lo) & (dv < hi)
            w = sv | (((dv - lo) * CB) << DST_SHIFT)
            csum = plsc.cumsum(jnp.where(m, 1, 0).astype(jnp.int32))
            pos = wpos_v + csum - 1
            plsc.store_scatter(buf_v, [pos], w, mask=m)
            return wpos_v + plsc.all_reduce_population_count(m)

        wpos_v = lax.fori_loop(0, CH // L, step, wpos_v)
        wpos = jnp.max(wpos_v)
        nflush = wpos // FLUSH

        def flushk(k, _):
            pltpu.sync_copy(
                buf_v.at[pl.ds(_m8(k * FLUSH), FLUSH)],
                packed_ref.at[pl.ds(_m8(wid * EPAD + flushbase + k * FLUSH),
                                    FLUSH)])
            return 0
        lax.fori_loop(0, nflush, flushk, 0)

        @pl.when(nflush > 0)
        def _shift():
            def mv(i, _):
                buf_v[pl.ds(i * L, L)] = (
                    buf_v[pl.ds(nflush * FLUSH + i * L, L)])
                return 0
            lax.fori_loop(0, FLUSH // L, mv, 0)

        wpos = wpos - nflush * FLUSH
        return (jnp.full((L,), wpos, jnp.int32),
                flushbase + nflush * FLUSH)

    wpos_v, flushbase = lax.fori_loop(
        0, NCHUNK, chunk, (jnp.zeros((L,), jnp.int32), jnp.int32(0)))
    wpos = jnp.max(wpos_v)
    k_cnt = flushbase + wpos

    iota = lax.iota(jnp.int32, L)
    sent = jnp.full((L,), SENT_WORD, jnp.int32)
    for k in range(GRP // L):
        plsc.store_scatter(buf_v, [wpos + k * L + iota], sent)
    n64 = (wpos + GRP - 1) // GRP

    def tailk(k, _):
        pltpu.sync_copy(
            buf_v.at[pl.ds(_m8(k * GRP), GRP)],
            packed_ref.at[pl.ds(_m8(wid * EPAD + flushbase + k * GRP), GRP)])
        return 0
    lax.fori_loop(0, n64, tailk, 0)

    cnt_v[...] = jnp.full((L,), k_cnt, jnp.int32)
    pltpu.sync_copy(cnt_v, counts_ref.at[pl.ds(_m8(wid * L), L)])


def _acc_body(hp_hbm, packed_ref, counts_ref, agg_out, *rest):
    aggs = rest[:NCB]
    (hp_sh, agg_full, st0, st1, wd0, wd1, ix0, ix1, cnt_v,
     sm0, sm1) = rest[NCB:]
    stage_v = (st0, st1)
    words_v = (wd0, wd1)
    idx_v = (ix0, ix1)
    sem = (sm0, sm1)

    wid = _wid()

    sid = lax.axis_index("s")

    @pl.when(sid < NSUB - 1)
    def _stage_main():
        off = _m16(sid * 640)
        pltpu.sync_copy(hp_hbm.at[pl.ds(off, 640)],
                        hp_sh.at[pl.ds(off, 640)])

    @pl.when(sid == NSUB - 1)
    def _stage_tail():
        off = _m16(sid * 640)
        pltpu.sync_copy(hp_hbm.at[pl.ds(off, N - 640 * (NSUB - 1))],
                        hp_sh.at[pl.ds(off, N - 640 * (NSUB - 1))])

    zv = jnp.zeros((CB,), jnp.bfloat16)

    def zrow(i, _):
        for c in range(NCB):
            aggs[c][pl.ds(i * CB, CB)] = zv
        return 0
    lax.fori_loop(0, NPT + 1, zrow, 0)
    plsc.subcore_barrier()

    pltpu.sync_copy(counts_ref.at[pl.ds(_m8(wid * L), L)], cnt_v)
    k_cnt = jnp.max(cnt_v[...])
    n_grp = (k_cnt + GRP - 1) // GRP

    def _launch(b, g):
        pltpu.sync_copy(packed_ref.at[pl.ds(_m8(wid * EPAD + g * GRP), GRP)],
                        words_v[b])
        for k in range(GRP // L):
            w = words_v[b][pl.ds(k * L, L)]
            idx_v[b][pl.ds(k * L, L)] = w & SRC_MASK
        pltpu.make_async_copy(hp_sh.at[idx_v[b]], stage_v[b], sem[b]).start()

    def _process(b):
        pltpu.make_async_copy(hp_sh.at[idx_v[b]], stage_v[b], sem[b]).wait()

        def quarter(k, _):
            wv = words_v[b][pl.ds(k * L, L)]
            dvec = lax.shift_right_logical(wv, DST_SHIFT)
            for j in range(L):
                d = dvec[j]
                e = k * L + j
                r = pl.ds(d, CB)
                for c in range(NCB):
                    sb = plsc.bitcast(stage_v[b][e, pl.ds(c * L, L)],
                                      jnp.bfloat16)
                    aggs[c][r] = jnp.maximum(aggs[c][r], sb)
            return 0
        lax.fori_loop(0, GRP // L, quarter, 0)

    @pl.when(n_grp > 0)
    def _pipeline():
        _launch(0, 0)

        def pair(p, _):
            g1 = 2 * p + 1
            g2 = 2 * p + 2

            @pl.when(g1 < n_grp)
            def _():
                _launch(1, g1)
            _process(0)

            @pl.when(g2 < n_grp)
            def _():
                _launch(0, g2)

            @pl.when(g1 < n_grp)
            def _():
                _process(1)
            return 0
        lax.fori_loop(0, (n_grp + 1) // 2, pair, 0)

    def arow(i, _):
        for c in range(NCB):
            agg_full[i, pl.ds(c * CB, CB)] = aggs[c][pl.ds(i * CB, CB)]
        return 0
    lax.fori_loop(0, NPT, arow, 0)

    pltpu.sync_copy(agg_full, agg_out.at[pl.ds(_m16(wid * NPT), NPT)])


_SC_MESH = plsc.VectorSubcoreMesh(core_axis_name="c", subcore_axis_name="s")

_ACC_SCRATCH = [
    pltpu.VMEM(((NPT + 1) * CB,), jnp.bfloat16) for _ in range(NCB)
] + [
    pltpu.VMEM_SHARED((N, D // 2), jnp.int32),
    pltpu.VMEM((NPT, D), jnp.bfloat16),
    pltpu.VMEM((GRP, D // 2), jnp.int32),
    pltpu.VMEM((GRP, D // 2), jnp.int32),
    pltpu.VMEM((GRP,), jnp.int32),
    pltpu.VMEM((GRP,), jnp.int32),
    pltpu.VMEM((GRP,), jnp.int32),
    pltpu.VMEM((GRP,), jnp.int32),
    pltpu.VMEM((L,), jnp.int32),
    pltpu.SemaphoreType.DMA,
    pltpu.SemaphoreType.DMA,
]

_SC_PARAMS = pltpu.CompilerParams(needs_layout_passes=False,
                                  use_tc_tiling_on_sc=False)

_build_lists = pl.kernel(
    _build_body,
    mesh=_SC_MESH,
    compiler_params=_SC_PARAMS,
    out_type=(
        jax.ShapeDtypeStruct((NW * EPAD,), jnp.int32),
        jax.ShapeDtypeStruct((NW * L,), jnp.int32),
    ),
    scratch_types=[
        pltpu.VMEM((BUFW,), jnp.int32),
        pltpu.VMEM((CH,), jnp.int32),
        pltpu.VMEM((CH,), jnp.int32),
        pltpu.VMEM((L,), jnp.int32),
    ],
)

_seg_max = pl.kernel(
    _acc_body,
    mesh=_SC_MESH,
    compiler_params=_SC_PARAMS,
    out_type=jax.ShapeDtypeStruct((NPAD, D), jnp.bfloat16),
    scratch_types=list(_ACC_SCRATCH),
)


_ROWS = 1000
_GRID = N // _ROWS


def _mm(a, w):
    return lax.dot_general(a, w, (((1,), (1,)), ((), ())),
                           preferred_element_type=jnp.float32)


def _ln(rst, g, be):
    mu = jnp.mean(rst, axis=-1, keepdims=True)
    var = jnp.mean((rst - mu) ** 2, axis=-1, keepdims=True)
    return (rst - mu) * lax.rsqrt(var + 1e-5) * g + be


def _tc_input_body(x_ref, wi_ref, bi_ref, wp_ref, bp_ref, h_ref, hp_ref):
    h = _mm(x_ref[...], wi_ref[...]) + bi_ref[...]
    h_ref[...] = h
    hp_ref[...] = jax.nn.relu(
        _mm(h, wp_ref[...]) + bp_ref[...]).astype(jnp.bfloat16)


def _tc_self_body(h_ref, ws_ref, bs_ref, o_ref):
    o_ref[...] = _mm(h_ref[...], ws_ref[...]) + bs_ref[...]


def _tc_combine_body(self_ref, agg_ref, wn_ref, bn_ref,
                     g_ref, be_ref, wp_ref, bp_ref, h_out, hp_out):
    agg = agg_ref[...].astype(jnp.float32)
    rst = self_ref[...] + _mm(agg, wn_ref[...]) + bn_ref[...]
    rst = jax.nn.relu(rst)
    hn = _ln(rst, g_ref[...], be_ref[...])
    h_out[...] = hn
    hp_out[...] = jax.nn.relu(
        _mm(hn, wp_ref[...]) + bp_ref[...]).astype(jnp.bfloat16)


def _tc_combine_final_body(self_ref, agg_ref, wn_ref, bn_ref,
                           g_ref, be_ref, o_ref):
    agg = agg_ref[...].astype(jnp.float32)
    rst = self_ref[...] + _mm(agg, wn_ref[...]) + bn_ref[...]
    o_ref[...] = _ln(rst, g_ref[...], be_ref[...])


_row_spec = pl.BlockSpec((_ROWS, D), lambda i: (i, 0))
_w_spec = pl.BlockSpec((D, D), lambda i: (0, 0))
_b_spec = pl.BlockSpec((1, D), lambda i: (0, 0))
_f32 = jnp.float32

_tc_input = pl.pallas_call(
    _tc_input_body,
    grid=(_GRID,),
    in_specs=[_row_spec, _w_spec, _b_spec, _w_spec, _b_spec],
    out_specs=[_row_spec, _row_spec],
    out_shape=[jax.ShapeDtypeStruct((N, D), _f32),
               jax.ShapeDtypeStruct((N, D), jnp.bfloat16)],
)

_tc_self = pl.pallas_call(
    _tc_self_body,
    grid=(_GRID,),
    in_specs=[_row_spec, _w_spec, _b_spec],
    out_specs=_row_spec,
    out_shape=jax.ShapeDtypeStruct((N, D), _f32),
)

_tc_combine = pl.pallas_call(
    _tc_combine_body,
    grid=(_GRID,),
    in_specs=[_row_spec, _row_spec, _w_spec, _b_spec,
              _b_spec, _b_spec, _w_spec, _b_spec],
    out_specs=[_row_spec, _row_spec],
    out_shape=[jax.ShapeDtypeStruct((N, D), _f32),
               jax.ShapeDtypeStruct((N, D), jnp.bfloat16)],
)

_tc_combine_final = pl.pallas_call(
    _tc_combine_final_body,
    grid=(_GRID,),
    in_specs=[_row_spec, _row_spec, _w_spec, _b_spec, _b_spec, _b_spec],
    out_specs=_row_spec,
    out_shape=jax.ShapeDtypeStruct((N, D), _f32),
)


def kernel(x, edge_index, W_in, b_in,
           Wp0, bp0, Ws0, bs0, Wn0, bn0, g0, be0,
           Wp1, bp1, Ws1, bs1, Wn1, bn1, g1, be1,
           Wp2, bp2, Ws2, bs2, Wn2, bn2, g2, be2):
    src = edge_index[0]
    dst = edge_index[1]
    r = lambda v: v.reshape(1, D)

    pk = lambda a: lax.bitcast_convert_type(
        a.reshape(N, D // 2, 2), jnp.int32)

    packed, counts = _build_lists(src, dst)
    h, hp = _tc_input(x, W_in, r(b_in), Wp0, r(bp0))
    hp = pk(hp)

    agg = _seg_max(hp, packed, counts)
    s0 = _tc_self(h, Ws0, r(bs0))
    h, hp = _tc_combine(s0, agg[:N], Wn0, r(bn0), r(g0), r(be0), Wp1, r(bp1))
    hp = pk(hp)

    agg = _seg_max(hp, packed, counts)
    s1 = _tc_self(h, Ws1, r(bs1))
    h, hp = _tc_combine(s1, agg[:N], Wn1, r(bn1), r(g1), r(be1), Wp2, r(bp2))
    hp = pk(hp)

    agg = _seg_max(hp, packed, counts)
    s2 = _tc_self(h, Ws2, r(bs2))
    return _tc_combine_final(s2, agg[:N], Wn2, r(bn2), r(g2), r(be2))


# speedup vs baseline: 1.8669x; 1.0776x over previous
import functools

import jax
import jax.numpy as jnp
from jax import lax
from jax.experimental import pallas as pl
from jax.experimental.pallas import tpu as pltpu
from jax.experimental.pallas import tpu_sc as plsc

N = 10000
E = 320000
D = 128

NCORES = 2
NSUB = 16
NW = NCORES * NSUB
L = 16

NPT = 320
NPAD = NPT * NW
CH = 6400
NCHUNK = E // CH
BUFW = 12288
FLUSH = 4096
GRP = 128
EPAD = E + GRP

CB = 32
NCB = D // CB

SRC_BITS = 14
SRC_MASK = (1 << SRC_BITS) - 1
DST_SHIFT = SRC_BITS
SENT_WORD = (NPT * CB) << DST_SHIFT


def _m8(v):
    return pl.multiple_of(v, 8)


def _m16(v):
    return pl.multiple_of(v, 16)


def _wid():
    return lax.axis_index("s") * NCORES + lax.axis_index("c")


def _build_body(src_hbm, dst_hbm, packed_ref, counts_ref,
                buf_v, src_v, dst_v, cnt_v):
    wid = _wid()
    lo = wid * NPT
    hi = lo + NPT

    def chunk(c, carry):
        wpos_v, flushbase = carry
        pltpu.sync_copy(src_hbm.at[pl.ds(_m8(c * CH), CH)], src_v)
        pltpu.sync_copy(dst_hbm.at[pl.ds(_m8(c * CH), CH)], dst_v)

        def step(i, wpos_v):
            sv = src_v[pl.ds(i * L, L)]
            dv = dst_v[pl.ds(i * L, L)]
            m = (dv >= lo) & (dv < hi)
            w = sv | (((dv - lo) * CB) << DST_SHIFT)
            csum = plsc.cumsum(jnp.where(m, 1, 0).astype(jnp.int32))
            pos = wpos_v + csum - 1
            plsc.store_scatter(buf_v, [pos], w, mask=m)
            return wpos_v + plsc.all_reduce_population_count(m)

        wpos_v = lax.fori_loop(0, CH // L, step, wpos_v)
        wpos = jnp.max(wpos_v)
        nflush = wpos // FLUSH

        def flushk(k, _):
            pltpu.sync_copy(
                buf_v.at[pl.ds(_m8(k * FLUSH), FLUSH)],
                packed_ref.at[pl.ds(_m8(wid * EPAD + flushbase + k * FLUSH),
                                    FLUSH)])
            return 0
        lax.fori_loop(0, nflush, flushk, 0)

        @pl.when(nflush > 0)
        def _shift():
            def mv(i, _):
                buf_v[pl.ds(i * L, L)] = (
                    buf_v[pl.ds(nflush * FLUSH + i * L, L)])
                return 0
            lax.fori_loop(0, FLUSH // L, mv, 0)

        wpos = wpos - nflush * FLUSH
        return (jnp.full((L,), wpos, jnp.int32),
                flushbase + nflush * FLUSH)

    wpos_v, flushbase = lax.fori_loop(
        0, NCHUNK, chunk, (jnp.zeros((L,), jnp.int32), jnp.int32(0)))
    wpos = jnp.max(wpos_v)
    k_cnt = flushbase + wpos

    iota = lax.iota(jnp.int32, L)
    sent = jnp.full((L,), SENT_WORD, jnp.int32)
    for k in range(GRP // L):
        plsc.store_scatter(buf_v, [wpos + k * L + iota], sent)
    n64 = (wpos + GRP - 1) // GRP

    def tailk(k, _):
        pltpu.sync_copy(
            buf_v.at[pl.ds(_m8(k * GRP), GRP)],
            packed_ref.at[pl.ds(_m8(wid * EPAD + flushbase + k * GRP), GRP)])
        return 0
    lax.fori_loop(0, n64, tailk, 0)

    cnt_v[...] = jnp.full((L,), k_cnt, jnp.int32)
    pltpu.sync_copy(cnt_v, counts_ref.at[pl.ds(_m8(wid * L), L)])


def _acc_body(hp_hbm, packed_ref, counts_ref, agg_out, *rest):
    aggs = rest[:NCB]
    (hp_sh, agg_full, st0, st1, wd0, wd1, ix0, ix1, cnt_v,
     sm0, sm1) = rest[NCB:]
    stage_v = (st0, st1)
    words_v = (wd0, wd1)
    idx_v = (ix0, ix1)
    sem = (sm0, sm1)

    wid = _wid()

    sid = lax.axis_index("s")

    @pl.when(sid < NSUB - 1)
    def _stage_main():
        off = _m16(sid * 640)
        pltpu.sync_copy(hp_hbm.at[pl.ds(off, 640)],
                        hp_sh.at[pl.ds(off, 640)])

    @pl.when(sid == NSUB - 1)
    def _stage_tail():
        off = _m16(sid * 640)
        pltpu.sync_copy(hp_hbm.at[pl.ds(off, N - 640 * (NSUB - 1))],
                        hp_sh.at[pl.ds(off, N - 640 * (NSUB - 1))])

    zv = jnp.zeros((CB,), jnp.bfloat16)

    def zrow(i, _):
        for c in range(NCB):
            aggs[c][pl.ds(i * CB, CB)] = zv
        return 0
    lax.fori_loop(0, NPT + 1, zrow, 0)
    plsc.subcore_barrier()

    pltpu.sync_copy(counts_ref.at[pl.ds(_m8(wid * L), L)], cnt_v)
    k_cnt = jnp.max(cnt_v[...])
    n_grp = (k_cnt + GRP - 1) // GRP

    def _launch(b, g):
        pltpu.sync_copy(packed_ref.at[pl.ds(_m8(wid * EPAD + g * GRP), GRP)],
                        words_v[b])
        for k in range(GRP // L):
            w = words_v[b][pl.ds(k * L, L)]
            idx_v[b][pl.ds(k * L, L)] = w & SRC_MASK
        pltpu.make_async_copy(hp_sh.at[idx_v[b]], stage_v[b], sem[b]).start()

    def _process(b):
        pltpu.make_async_copy(hp_sh.at[idx_v[b]], stage_v[b], sem[b]).wait()

        def quarter(k, _):
            wv = words_v[b][pl.ds(k * L, L)]
            dvec = lax.shift_right_logical(wv, DST_SHIFT)
            for j in range(L):
                d = dvec[j]
                e = k * L + j
                r = pl.ds(d, CB)
                for c in range(NCB):
                    sb = plsc.bitcast(stage_v[b][e, pl.ds(c * L, L)],
                                      jnp.bfloat16)
                    aggs[c][r] = jnp.maximum(aggs[c][r], sb)
            return 0
        lax.fori_loop(0, GRP // L, quarter, 0)

    @pl.when(n_grp > 0)
    def _pipeline():
        _launch(0, 0)

        def pair(p, _):
            g1 = 2 * p + 1
            g2 = 2 * p + 2

            @pl.when(g1 < n_grp)
            def _():
                _launch(1, g1)
            _process(0)

            @pl.when(g2 < n_grp)
            def _():
                _launch(0, g2)

            @pl.when(g1 < n_grp)
            def _():
                _process(1)
            return 0
        lax.fori_loop(0, (n_grp + 1) // 2, pair, 0)

    def arow(i, _):
        for c in range(NCB):
            agg_full[i, pl.ds(c * CB, CB)] = aggs[c][pl.ds(i * CB, CB)]
        return 0
    lax.fori_loop(0, NPT, arow, 0)

    pltpu.sync_copy(agg_full, agg_out.at[pl.ds(_m16(wid * NPT), NPT)])


_SC_MESH = plsc.VectorSubcoreMesh(core_axis_name="c", subcore_axis_name="s")

_ACC_SCRATCH = [
    pltpu.VMEM(((NPT + 1) * CB,), jnp.bfloat16) for _ in range(NCB)
] + [
    pltpu.VMEM_SHARED((N, D // 2), jnp.int32),
    pltpu.VMEM((NPT, D), jnp.bfloat16),
    pltpu.VMEM((GRP, D // 2), jnp.int32),
    pltpu.VMEM((GRP, D // 2), jnp.int32),
    pltpu.VMEM((GRP,), jnp.int32),
    pltpu.VMEM((GRP,), jnp.int32),
    pltpu.VMEM((GRP,), jnp.int32),
    pltpu.VMEM((GRP,), jnp.int32),
    pltpu.VMEM((L,), jnp.int32),
    pltpu.SemaphoreType.DMA,
    pltpu.SemaphoreType.DMA,
]

_SC_PARAMS = pltpu.CompilerParams(needs_layout_passes=False,
                                  use_tc_tiling_on_sc=False)

_build_lists = pl.kernel(
    _build_body,
    mesh=_SC_MESH,
    compiler_params=_SC_PARAMS,
    out_type=(
        jax.ShapeDtypeStruct((NW * EPAD,), jnp.int32),
        jax.ShapeDtypeStruct((NW * L,), jnp.int32),
    ),
    scratch_types=[
        pltpu.VMEM((BUFW,), jnp.int32),
        pltpu.VMEM((CH,), jnp.int32),
        pltpu.VMEM((CH,), jnp.int32),
        pltpu.VMEM((L,), jnp.int32),
    ],
)

_seg_max = pl.kernel(
    _acc_body,
    mesh=_SC_MESH,
    compiler_params=_SC_PARAMS,
    out_type=jax.ShapeDtypeStruct((NPAD, D), jnp.bfloat16),
    scratch_types=list(_ACC_SCRATCH),
)


_ROWS = 1000
_GRID = N // _ROWS


def _mm(a, w):
    return lax.dot_general(a, w, (((1,), (1,)), ((), ())),
                           preferred_element_type=jnp.float32)


def _ln(rst, g, be):
    mu = jnp.mean(rst, axis=-1, keepdims=True)
    var = jnp.mean((rst - mu) ** 2, axis=-1, keepdims=True)
    return (rst - mu) * lax.rsqrt(var + 1e-5) * g + be


def _tc_input_body(x_ref, wi_ref, bi_ref, wp_ref, bp_ref, h_ref, hp_ref):
    h = _mm(x_ref[...], wi_ref[...]) + bi_ref[...]
    h_ref[...] = h
    hp_ref[...] = jax.nn.relu(
        _mm(h, wp_ref[...]) + bp_ref[...]).astype(jnp.bfloat16)


def _tc_self_body(h_ref, ws_ref, bs_ref, o_ref):
    o_ref[...] = _mm(h_ref[...], ws_ref[...]) + bs_ref[...]


def _tc_combine_body(self_ref, agg_ref, wn_ref, bn_ref,
                     g_ref, be_ref, wp_ref, bp_ref, h_out, hp_out):
    agg = agg_ref[...].astype(jnp.float32)
    rst = self_ref[...] + _mm(agg, wn_ref[...]) + bn_ref[...]
    rst = jax.nn.relu(rst)
    hn = _ln(rst, g_ref[...], be_ref[...])
    h_out[...] = hn
    hp_out[...] = jax.nn.relu(
        _mm(hn, wp_ref[...]) + bp_ref[...]).astype(jnp.bfloat16)


def _tc_combine_final_body(self_ref, agg_ref, wn_ref, bn_ref,
                           g_ref, be_ref, o_ref):
    agg = agg_ref[...].astype(jnp.float32)
    rst = self_ref[...] + _mm(agg, wn_ref[...]) + bn_ref[...]
    o_ref[...] = _ln(rst, g_ref[...], be_ref[...])


_row_spec = pl.BlockSpec((_ROWS, D), lambda i: (i, 0))
_w_spec = pl.BlockSpec((D, D), lambda i: (0, 0))
_b_spec = pl.BlockSpec((1, D), lambda i: (0, 0))
_f32 = jnp.float32

_tc_input = pl.pallas_call(
    _tc_input_body,
    grid=(_GRID,),
    in_specs=[_row_spec, _w_spec, _b_spec, _w_spec, _b_spec],
    out_specs=[_row_spec, _row_spec],
    out_shape=[jax.ShapeDtypeStruct((N, D), _f32),
               jax.ShapeDtypeStruct((N, D), jnp.bfloat16)],
)

_tc_self = pl.pallas_call(
    _tc_self_body,
    grid=(_GRID,),
    in_specs=[_row_spec, _w_spec, _b_spec],
    out_specs=_row_spec,
    out_shape=jax.ShapeDtypeStruct((N, D), _f32),
)

_tc_combine = pl.pallas_call(
    _tc_combine_body,
    grid=(_GRID,),
    in_specs=[_row_spec, _row_spec, _w_spec, _b_spec,
              _b_spec, _b_spec, _w_spec, _b_spec],
    out_specs=[_row_spec, _row_spec],
    out_shape=[jax.ShapeDtypeStruct((N, D), _f32),
               jax.ShapeDtypeStruct((N, D), jnp.bfloat16)],
)

_tc_combine_final = pl.pallas_call(
    _tc_combine_final_body,
    grid=(_GRID,),
    in_specs=[_row_spec, _row_spec, _w_spec, _b_spec, _b_spec, _b_spec],
    out_specs=_row_spec,
    out_shape=jax.ShapeDtypeStruct((N, D), _f32),
)


def kernel(x, edge_index, W_in, b_in,
           Wp0, bp0, Ws0, bs0, Wn0, bn0, g0, be0,
           Wp1, bp1, Ws1, bs1, Wn1, bn1, g1, be1,
           Wp2, bp2, Ws2, bs2, Wn2, bn2, g2, be2):
    src = edge_index[0]
    dst = edge_index[1]
    r = lambda v: v.reshape(1, D)

    pk = lambda a: lax.bitcast_convert_type(
        a.reshape(N, D // 2, 2), jnp.int32)

    packed, counts = _build_lists(src, dst)
    h, hp = _tc_input(x, W_in, r(b_in), Wp0, r(bp0))
    hp = pk(hp)

    agg = _seg_max(hp, packed, counts)
    s0 = _tc_self(h, Ws0, r(bs0))
    h, hp = _tc_combine(s0, agg[:N], Wn0, r(bn0), r(g0), r(be0), Wp1, r(bp1))
    hp = pk(hp)

    agg = _seg_max(hp, packed, counts)
    s1 = _tc_self(h, Ws1, r(bs1))
    h, hp = _tc_combine(s1, agg[:N], Wn1, r(bn1), r(g1), r(be1), Wp2, r(bp2))
    hp = pk(hp)

    agg = _seg_max(hp, packed, counts)
    s2 = _tc_self(h, Ws2, r(bs2))
    return _tc_combine_final(s2, agg[:N], Wn2, r(bn2), r(g2), r(be2))


# speedup vs baseline: 1.9530x; 1.0461x over previous
import functools

import jax
import jax.numpy as jnp
from jax import lax
from jax.experimental import pallas as pl
from jax.experimental.pallas import tpu as pltpu
from jax.experimental.pallas import tpu_sc as plsc

N = 10000
E = 320000
D = 128

NCORES = 2
NSUB = 16
NW = NCORES * NSUB
L = 16

NPT = 320
NPAD = NPT * NW
CH = 6400
NCHUNK = E // CH
BUFW = 12288
FLUSH = 4096
GRP = 256
EPAD = E + GRP

CB = 32
NCB = D // CB

SRC_BITS = 14
SRC_MASK = (1 << SRC_BITS) - 1
DST_SHIFT = SRC_BITS
SENT_WORD = (NPT * CB) << DST_SHIFT


def _m8(v):
    return pl.multiple_of(v, 8)


def _m16(v):
    return pl.multiple_of(v, 16)


def _wid():
    return lax.axis_index("s") * NCORES + lax.axis_index("c")


def _build_body(src_hbm, dst_hbm, packed_ref, counts_ref,
                buf_v, src_v, dst_v, cnt_v):
    wid = _wid()
    lo = wid * NPT
    hi = lo + NPT

    def chunk(c, carry):
        wpos_v, flushbase = carry
        pltpu.sync_copy(src_hbm.at[pl.ds(_m8(c * CH), CH)], src_v)
        pltpu.sync_copy(dst_hbm.at[pl.ds(_m8(c * CH), CH)], dst_v)

        def step(i, wpos_v):
            sv = src_v[pl.ds(i * L, L)]
            dv = dst_v[pl.ds(i * L, L)]
            m = (dv >= lo) & (dv < hi)
            w = sv | (((dv - lo) * CB) << DST_SHIFT)
            csum = plsc.cumsum(jnp.where(m, 1, 0).astype(jnp.int32))
            pos = wpos_v + csum - 1
            plsc.store_scatter(buf_v, [pos], w, mask=m)
            return wpos_v + plsc.all_reduce_population_count(m)

        wpos_v = lax.fori_loop(0, CH // L, step, wpos_v)
        wpos = jnp.max(wpos_v)
        nflush = wpos // FLUSH

        def flushk(k, _):
            pltpu.sync_copy(
                buf_v.at[pl.ds(_m8(k * FLUSH), FLUSH)],
                packed_ref.at[pl.ds(_m8(wid * EPAD + flushbase + k * FLUSH),
                                    FLUSH)])
            return 0
        lax.fori_loop(0, nflush, flushk, 0)

        @pl.when(nflush > 0)
        def _shift():
            def mv(i, _):
                buf_v[pl.ds(i * L, L)] = (
                    buf_v[pl.ds(nflush * FLUSH + i * L, L)])
                return 0
            lax.fori_loop(0, FLUSH // L, mv, 0)

        wpos = wpos - nflush * FLUSH
        return (jnp.full((L,), wpos, jnp.int32),
                flushbase + nflush * FLUSH)

    wpos_v, flushbase = lax.fori_loop(
        0, NCHUNK, chunk, (jnp.zeros((L,), jnp.int32), jnp.int32(0)))
    wpos = jnp.max(wpos_v)
    k_cnt = flushbase + wpos

    iota = lax.iota(jnp.int32, L)
    sent = jnp.full((L,), SENT_WORD, jnp.int32)
    for k in range(GRP // L):
        plsc.store_scatter(buf_v, [wpos + k * L + iota], sent)
    n64 = (wpos + GRP - 1) // GRP

    def tailk(k, _):
        pltpu.sync_copy(
            buf_v.at[pl.ds(_m8(k * GRP), GRP)],
            packed_ref.at[pl.ds(_m8(wid * EPAD + flushbase + k * GRP), GRP)])
        return 0
    lax.fori_loop(0, n64, tailk, 0)

    cnt_v[...] = jnp.full((L,), k_cnt, jnp.int32)
    pltpu.sync_copy(cnt_v, counts_ref.at[pl.ds(_m8(wid * L), L)])


def _acc_body(hp_hbm, packed_ref, counts_ref, agg_out, *rest):
    aggs = rest[:NCB]
    (hp_sh, agg_full, st0, st1, wd0, wd1, ix0, ix1, cnt_v,
     sm0, sm1) = rest[NCB:]
    stage_v = (st0, st1)
    words_v = (wd0, wd1)
    idx_v = (ix0, ix1)
    sem = (sm0, sm1)

    wid = _wid()

    sid = lax.axis_index("s")

    @pl.when(sid < NSUB - 1)
    def _stage_main():
        off = _m16(sid * 640)
        pltpu.sync_copy(hp_hbm.at[pl.ds(off, 640)],
                        hp_sh.at[pl.ds(off, 640)])

    @pl.when(sid == NSUB - 1)
    def _stage_tail():
        off = _m16(sid * 640)
        pltpu.sync_copy(hp_hbm.at[pl.ds(off, N - 640 * (NSUB - 1))],
                        hp_sh.at[pl.ds(off, N - 640 * (NSUB - 1))])

    zv = jnp.zeros((CB,), jnp.bfloat16)

    def zrow(i, _):
        for c in range(NCB):
            aggs[c][pl.ds(i * CB, CB)] = zv
        return 0
    lax.fori_loop(0, NPT + 1, zrow, 0)
    plsc.subcore_barrier()

    pltpu.sync_copy(counts_ref.at[pl.ds(_m8(wid * L), L)], cnt_v)
    k_cnt = jnp.max(cnt_v[...])
    n_grp = (k_cnt + GRP - 1) // GRP

    def _launch(b, g):
        pltpu.sync_copy(packed_ref.at[pl.ds(_m8(wid * EPAD + g * GRP), GRP)],
                        words_v[b])
        for k in range(GRP // L):
            w = words_v[b][pl.ds(k * L, L)]
            idx_v[b][pl.ds(k * L, L)] = w & SRC_MASK
        pltpu.make_async_copy(hp_sh.at[idx_v[b]], stage_v[b], sem[b]).start()

    def _process(b):
        pltpu.make_async_copy(hp_sh.at[idx_v[b]], stage_v[b], sem[b]).wait()

        def quarter(k, _):
            wv = words_v[b][pl.ds(k * L, L)]
            dvec = lax.shift_right_logical(wv, DST_SHIFT)
            for j in range(L):
                d = dvec[j]
                e = k * L + j
                r = pl.ds(d, CB)
                for c in range(NCB):
                    sb = plsc.bitcast(stage_v[b][e, pl.ds(c * L, L)],
                                      jnp.bfloat16)
                    aggs[c][r] = jnp.maximum(aggs[c][r], sb)
            return 0
        lax.fori_loop(0, GRP // L, quarter, 0)

    @pl.when(n_grp > 0)
    def _pipeline():
        _launch(0, 0)

        def pair(p, _):
            g1 = 2 * p + 1
            g2 = 2 * p + 2

            @pl.when(g1 < n_grp)
            def _():
                _launch(1, g1)
            _process(0)

            @pl.when(g2 < n_grp)
            def _():
                _launch(0, g2)

            @pl.when(g1 < n_grp)
            def _():
                _process(1)
            return 0
        lax.fori_loop(0, (n_grp + 1) // 2, pair, 0)

    def arow(i, _):
        for c in range(NCB):
            agg_full[i, pl.ds(c * CB, CB)] = aggs[c][pl.ds(i * CB, CB)]
        return 0
    lax.fori_loop(0, NPT, arow, 0)

    pltpu.sync_copy(agg_full, agg_out.at[pl.ds(_m16(wid * NPT), NPT)])


_SC_MESH = plsc.VectorSubcoreMesh(core_axis_name="c", subcore_axis_name="s")

_ACC_SCRATCH = [
    pltpu.VMEM(((NPT + 1) * CB,), jnp.bfloat16) for _ in range(NCB)
] + [
    pltpu.VMEM_SHARED((N, D // 2), jnp.int32),
    pltpu.VMEM((NPT, D), jnp.bfloat16),
    pltpu.VMEM((GRP, D // 2), jnp.int32),
    pltpu.VMEM((GRP, D // 2), jnp.int32),
    pltpu.VMEM((GRP,), jnp.int32),
    pltpu.VMEM((GRP,), jnp.int32),
    pltpu.VMEM((GRP,), jnp.int32),
    pltpu.VMEM((GRP,), jnp.int32),
    pltpu.VMEM((L,), jnp.int32),
    pltpu.SemaphoreType.DMA,
    pltpu.SemaphoreType.DMA,
]

_SC_PARAMS = pltpu.CompilerParams(needs_layout_passes=False,
                                  use_tc_tiling_on_sc=False)

_build_lists = pl.kernel(
    _build_body,
    mesh=_SC_MESH,
    compiler_params=_SC_PARAMS,
    out_type=(
        jax.ShapeDtypeStruct((NW * EPAD,), jnp.int32),
        jax.ShapeDtypeStruct((NW * L,), jnp.int32),
    ),
    scratch_types=[
        pltpu.VMEM((BUFW,), jnp.int32),
        pltpu.VMEM((CH,), jnp.int32),
        pltpu.VMEM((CH,), jnp.int32),
        pltpu.VMEM((L,), jnp.int32),
    ],
)

_seg_max = pl.kernel(
    _acc_body,
    mesh=_SC_MESH,
    compiler_params=_SC_PARAMS,
    out_type=jax.ShapeDtypeStruct((NPAD, D), jnp.bfloat16),
    scratch_types=list(_ACC_SCRATCH),
)


_ROWS = 1000
_GRID = N // _ROWS


def _mm(a, w):
    return lax.dot_general(a, w, (((1,), (1,)), ((), ())),
                           preferred_element_type=jnp.float32)


def _ln(rst, g, be):
    mu = jnp.mean(rst, axis=-1, keepdims=True)
    var = jnp.mean((rst - mu) ** 2, axis=-1, keepdims=True)
    return (rst - mu) * lax.rsqrt(var + 1e-5) * g + be


def _tc_input_body(x_ref, wi_ref, bi_ref, wp_ref, bp_ref, h_ref, hp_ref):
    h = _mm(x_ref[...], wi_ref[...]) + bi_ref[...]
    h_ref[...] = h
    hp_ref[...] = jax.nn.relu(
        _mm(h, wp_ref[...]) + bp_ref[...]).astype(jnp.bfloat16)


def _tc_self_body(h_ref, ws_ref, bs_ref, o_ref):
    o_ref[...] = _mm(h_ref[...], ws_ref[...]) + bs_ref[...]


def _tc_combine_body(self_ref, agg_ref, wn_ref, bn_ref,
                     g_ref, be_ref, wp_ref, bp_ref, h_out, hp_out):
    agg = agg_ref[...].astype(jnp.float32)
    rst = self_ref[...] + _mm(agg, wn_ref[...]) + bn_ref[...]
    rst = jax.nn.relu(rst)
    hn = _ln(rst, g_ref[...], be_ref[...])
    h_out[...] = hn
    hp_out[...] = jax.nn.relu(
        _mm(hn, wp_ref[...]) + bp_ref[...]).astype(jnp.bfloat16)


def _tc_combine_final_body(self_ref, agg_ref, wn_ref, bn_ref,
                           g_ref, be_ref, o_ref):
    agg = agg_ref[...].astype(jnp.float32)
    rst = self_ref[...] + _mm(agg, wn_ref[...]) + bn_ref[...]
    o_ref[...] = _ln(rst, g_ref[...], be_ref[...])


_row_spec = pl.BlockSpec((_ROWS, D), lambda i: (i, 0))
_w_spec = pl.BlockSpec((D, D), lambda i: (0, 0))
_b_spec = pl.BlockSpec((1, D), lambda i: (0, 0))
_f32 = jnp.float32

_tc_input = pl.pallas_call(
    _tc_input_body,
    grid=(_GRID,),
    in_specs=[_row_spec, _w_spec, _b_spec, _w_spec, _b_spec],
    out_specs=[_row_spec, _row_spec],
    out_shape=[jax.ShapeDtypeStruct((N, D), _f32),
               jax.ShapeDtypeStruct((N, D), jnp.bfloat16)],
)

_tc_self = pl.pallas_call(
    _tc_self_body,
    grid=(_GRID,),
    in_specs=[_row_spec, _w_spec, _b_spec],
    out_specs=_row_spec,
    out_shape=jax.ShapeDtypeStruct((N, D), _f32),
)

_tc_combine = pl.pallas_call(
    _tc_combine_body,
    grid=(_GRID,),
    in_specs=[_row_spec, _row_spec, _w_spec, _b_spec,
              _b_spec, _b_spec, _w_spec, _b_spec],
    out_specs=[_row_spec, _row_spec],
    out_shape=[jax.ShapeDtypeStruct((N, D), _f32),
               jax.ShapeDtypeStruct((N, D), jnp.bfloat16)],
)

_tc_combine_final = pl.pallas_call(
    _tc_combine_final_body,
    grid=(_GRID,),
    in_specs=[_row_spec, _row_spec, _w_spec, _b_spec, _b_spec, _b_spec],
    out_specs=_row_spec,
    out_shape=jax.ShapeDtypeStruct((N, D), _f32),
)


def kernel(x, edge_index, W_in, b_in,
           Wp0, bp0, Ws0, bs0, Wn0, bn0, g0, be0,
           Wp1, bp1, Ws1, bs1, Wn1, bn1, g1, be1,
           Wp2, bp2, Ws2, bs2, Wn2, bn2, g2, be2):
    src = edge_index[0]
    dst = edge_index[1]
    r = lambda v: v.reshape(1, D)

    pk = lambda a: lax.bitcast_convert_type(
        a.reshape(N, D // 2, 2), jnp.int32)

    packed, counts = _build_lists(src, dst)
    h, hp = _tc_input(x, W_in, r(b_in), Wp0, r(bp0))
    hp = pk(hp)

    agg = _seg_max(hp, packed, counts)
    s0 = _tc_self(h, Ws0, r(bs0))
    h, hp = _tc_combine(s0, agg[:N], Wn0, r(bn0), r(g0), r(be0), Wp1, r(bp1))
    hp = pk(hp)

    agg = _seg_max(hp, packed, counts)
    s1 = _tc_self(h, Ws1, r(bs1))
    h, hp = _tc_combine(s1, agg[:N], Wn1, r(bn1), r(g1), r(be1), Wp2, r(bp2))
    hp = pk(hp)

    agg = _seg_max(hp, packed, counts)
    s2 = _tc_self(h, Ws2, r(bs2))
    return _tc_combine_final(s2, agg[:N], Wn2, r(bn2), r(g2), r(be2))
